# Initial kernel scaffold; baseline (speedup 1.0000x reference)
import jax
import jax.numpy as jnp
from jax.experimental import pallas as pl


def kernel(x, edge_index, W):
    raise NotImplementedError("write your pallas kernel here")


# speedup vs baseline: 25.1394x; 25.1394x over previous
import functools

import jax
import jax.numpy as jnp
from jax import lax
from jax.experimental import pallas as pl
from jax.experimental.pallas import tpu as pltpu
from jax.experimental.pallas import tpu_sc as plsc

N_NODES = 10000
N_EDGES = 320000
D = 128

NP = 10240
NC, NS = 2, 16
NW = NC * NS
EPW = N_EDGES // NW
CH = 80
NCH = EPW // CH
NPT = NP // NS

_MESH = plsc.VectorSubcoreMesh(
    core_axis_name="c", subcore_axis_name="s", num_cores=NC, num_subcores=NS
)


@functools.partial(
    pl.kernel,
    out_type=jax.ShapeDtypeStruct((NC, NP), jnp.float32),
    mesh=_MESH,
    scratch_types=[
        pltpu.VMEM((NCH, CH), jnp.int32),
        pltpu.VMEM((NPT,), jnp.float32),
        pltpu.VMEM((CH,), jnp.float32),
        pltpu.VMEM_SHARED((NP,), jnp.float32),
    ],
)
def _deg_kernel(row_hbm, out_hbm, row_v, zbuf, ones_v, hist_sh):
    c = lax.axis_index("c")
    s = lax.axis_index("s")
    wid = s * NC + c

    @pl.loop(0, NPT // 16)
    def _zero(i):
        zbuf[pl.ds(i * 16, 16)] = jnp.zeros((16,), jnp.float32)

    @pl.loop(0, CH // 16)
    def _one(i):
        ones_v[pl.ds(i * 16, 16)] = jnp.ones((16,), jnp.float32)

    pltpu.sync_copy(zbuf, hist_sh.at[pl.ds(s * NPT, NPT)])
    pltpu.sync_copy(row_hbm.at[wid], row_v)
    plsc.subcore_barrier()

    @pl.loop(0, NCH)
    def _hist(j):
        pltpu.sync_copy(ones_v, hist_sh.at[row_v.at[j]], add=True)

    plsc.subcore_barrier()
    pltpu.sync_copy(hist_sh.at[pl.ds(s * NPT, NPT)],
                    out_hbm.at[c, pl.ds(s * NPT, NPT)])


@functools.partial(
    pl.kernel,
    out_type=jax.ShapeDtypeStruct((NC, NP, D), jnp.float32),
    mesh=_MESH,
    scratch_types=[
        pltpu.VMEM((NCH, CH), jnp.int32),
        pltpu.VMEM((NCH, CH), jnp.int32),
        pltpu.VMEM((CH, D), jnp.float32),
        pltpu.VMEM_SHARED((NP, D), jnp.float32),
    ],
)
def _agg_kernel(y_hbm, col_hbm, row_hbm, out_hbm,
                col_v, row_v, gbuf, s_sh):
    c = lax.axis_index("c")
    s = lax.axis_index("s")
    wid = s * NC + c

    @pl.loop(0, CH * (D // 16))
    def _zero(i):
        gbuf[i // (D // 16), pl.ds((i % (D // 16)) * 16, 16)] = (
            jnp.zeros((16,), jnp.float32))

    @pl.loop(0, NPT // CH)
    def _zinit(k):
        pltpu.sync_copy(gbuf, s_sh.at[pl.ds(s * NPT + k * CH, CH)])

    pltpu.sync_copy(col_hbm.at[wid], col_v)
    pltpu.sync_copy(row_hbm.at[wid], row_v)
    plsc.subcore_barrier()

    @pl.loop(0, NCH)
    def _edges(j):
        pltpu.sync_copy(y_hbm.at[col_v.at[j]], gbuf)
        pltpu.sync_copy(gbuf, s_sh.at[row_v.at[j]], add=True)

    plsc.subcore_barrier()
    pltpu.sync_copy(s_sh.at[pl.ds(s * NPT, NPT)],
                    out_hbm.at[c, pl.ds(s * NPT, NPT)])


def _scale_body(h0_ref, h1_ref, x_ref, ybar_ref, dinv_ref):
    deg = h0_ref[...] + h1_ref[...] + 1.0
    dinv = lax.rsqrt(deg)
    dinv_ref[...] = dinv
    ybar_ref[...] = dinv * x_ref[...]


_scale_call = pl.pallas_call(
    _scale_body,
    grid=(NP // 1024,),
    in_specs=[
        pl.BlockSpec((1024, 1), lambda i: (i, 0)),
        pl.BlockSpec((1024, 1), lambda i: (i, 0)),
        pl.BlockSpec((1024, D), lambda i: (i, 0)),
    ],
    out_specs=[
        pl.BlockSpec((1024, D), lambda i: (i, 0)),
        pl.BlockSpec((1024, 1), lambda i: (i, 0)),
    ],
    out_shape=[
        jax.ShapeDtypeStruct((NP, D), jnp.float32),
        jax.ShapeDtypeStruct((NP, 1), jnp.float32),
    ],
)


def _final_body(s0_ref, s1_ref, ybar_ref, dinv_ref, w_ref, out_ref):
    z = (s0_ref[...] + s1_ref[...] + ybar_ref[...]) * dinv_ref[...]
    out_ref[...] = lax.dot_general(
        z, w_ref[...], (((1,), (1,)), ((), ())),
        preferred_element_type=jnp.float32)


_final_call = pl.pallas_call(
    _final_body,
    grid=(NP // 1024,),
    in_specs=[
        pl.BlockSpec((1024, D), lambda i: (i, 0)),
        pl.BlockSpec((1024, D), lambda i: (i, 0)),
        pl.BlockSpec((1024, D), lambda i: (i, 0)),
        pl.BlockSpec((1024, 1), lambda i: (i, 0)),
        pl.BlockSpec((D, D), lambda i: (0, 0)),
    ],
    out_specs=pl.BlockSpec((1024, D), lambda i: (i, 0)),
    out_shape=jax.ShapeDtypeStruct((NP, D), jnp.float32),
)


def kernel(x, edge_index, W):
    ei = edge_index.astype(jnp.int32)
    row2d = ei[0].reshape(NW, NCH, CH)
    col2d = ei[1].reshape(NW, NCH, CH)
    x_pad = jnp.pad(x, ((0, NP - N_NODES), (0, 0)))

    hist = _deg_kernel(row2d)
    ybar, dinv = _scale_call(hist[0].reshape(NP, 1), hist[1].reshape(NP, 1),
                             x_pad)
    spart = _agg_kernel(ybar, col2d, row2d)
    outp = _final_call(spart[0], spart[1], ybar, dinv, W)
    return outp[:N_NODES]


# speedup vs baseline: 35.7211x; 1.4209x over previous
import functools

import jax
import jax.numpy as jnp
from jax import lax
from jax.experimental import pallas as pl
from jax.experimental.pallas import tpu as pltpu
from jax.experimental.pallas import tpu_sc as plsc

N_NODES = 10000
N_EDGES = 320000
D = 128

NP = 10240
NC, NS = 2, 16
NW = NC * NS
EPW = N_EDGES // NW
CH = 80
NCH = EPW // CH
NPT = NP // NS

_MESH = plsc.VectorSubcoreMesh(
    core_axis_name="c", subcore_axis_name="s", num_cores=NC, num_subcores=NS
)


@functools.partial(
    pl.kernel,
    out_type=jax.ShapeDtypeStruct((NC, NP), jnp.float32),
    mesh=_MESH,
    scratch_types=[
        pltpu.VMEM((NCH, CH), jnp.int32),
        pltpu.VMEM((NPT,), jnp.float32),
        pltpu.VMEM((CH,), jnp.float32),
        pltpu.VMEM_SHARED((NP,), jnp.float32),
    ],
)
def _deg_kernel(row_hbm, out_hbm, row_v, zbuf, ones_v, hist_sh):
    c = lax.axis_index("c")
    s = lax.axis_index("s")
    wid = s * NC + c

    @pl.loop(0, NPT // 16)
    def _zero(i):
        zbuf[pl.ds(i * 16, 16)] = jnp.zeros((16,), jnp.float32)

    @pl.loop(0, CH // 16)
    def _one(i):
        ones_v[pl.ds(i * 16, 16)] = jnp.ones((16,), jnp.float32)

    pltpu.sync_copy(zbuf, hist_sh.at[pl.ds(s * NPT, NPT)])
    pltpu.sync_copy(row_hbm.at[wid], row_v)
    plsc.subcore_barrier()

    @pl.loop(0, NCH)
    def _hist(j):
        pltpu.sync_copy(ones_v, hist_sh.at[row_v.at[j]], add=True)

    plsc.subcore_barrier()
    pltpu.sync_copy(hist_sh.at[pl.ds(s * NPT, NPT)],
                    out_hbm.at[c, pl.ds(s * NPT, NPT)])


@functools.partial(
    pl.kernel,
    out_type=jax.ShapeDtypeStruct((NC, NP, D), jnp.float32),
    mesh=_MESH,
    scratch_types=[
        pltpu.VMEM((EPW,), jnp.int32),
        pltpu.VMEM((NCH, CH), jnp.int32),
        pltpu.VMEM((2, CH, D), jnp.float32),
        pltpu.VMEM_SHARED((NP, D), jnp.float32),
        pltpu.SemaphoreType.DMA,
        pltpu.SemaphoreType.DMA,
        pltpu.SemaphoreType.DMA,
        pltpu.SemaphoreType.DMA,
    ],
)
def _agg_kernel(y_hbm, col_hbm, row_hbm, out_hbm,
                col_v, row_v, gbuf, s_sh, gsem0, gsem1, ssem0, ssem1):
    c = lax.axis_index("c")
    s = lax.axis_index("s")
    wid = s * NC + c
    gsem = (gsem0, gsem1)
    ssem = (ssem0, ssem1)

    @pl.loop(0, CH * (D // 16))
    def _zero(i):
        gbuf[0, i // (D // 16), pl.ds((i % (D // 16)) * 16, 16)] = (
            jnp.zeros((16,), jnp.float32))

    @pl.loop(0, NPT // CH)
    def _zinit(k):
        pltpu.sync_copy(gbuf.at[0], s_sh.at[pl.ds(s * NPT + k * CH, CH)])

    pltpu.sync_copy(col_hbm.at[wid], col_v)
    pltpu.sync_copy(row_hbm.at[wid], row_v)
    plsc.subcore_barrier()

    def start_gather(j, b):
        pltpu.async_copy(y_hbm.at[col_v.at[pl.ds(j * CH, CH)]], gbuf.at[b],
                         gsem[b])

    def wait_gather(j, b):
        pltpu.make_async_copy(y_hbm.at[col_v.at[pl.ds(j * CH, CH)]],
                              gbuf.at[b], gsem[b]).wait()

    def start_scatter(j, b):
        pltpu.async_copy(gbuf.at[b], s_sh.at[row_v.at[j]], ssem[b], add=True)

    def wait_scatter(j, b):
        pltpu.make_async_copy(gbuf.at[b], s_sh.at[row_v.at[j]],
                              ssem[b]).wait()

    start_gather(0, 0)
    start_gather(1, 1)
    wait_gather(0, 0)
    start_scatter(0, 0)

    @pl.loop(1, NCH // 2)
    def _edges(t):
        for b in (0, 1):
            j = t * 2 + b
            wait_scatter(j - 2, b)
            start_gather(j, b)
            wait_gather(j - 1, 1 - b)
            start_scatter(j - 1, 1 - b)

    wait_scatter(NCH - 3, 0)
    start_gather(NCH - 1, 0)
    wait_gather(NCH - 2, 1)
    start_scatter(NCH - 2, 1)
    wait_gather(NCH - 1, 0)
    start_scatter(NCH - 1, 0)
    wait_scatter(NCH - 2, 1)
    wait_scatter(NCH - 1, 0)

    plsc.subcore_barrier()
    pltpu.sync_copy(s_sh.at[pl.ds(s * NPT, NPT)],
                    out_hbm.at[c, pl.ds(s * NPT, NPT)])


def _scale_body(h0_ref, h1_ref, x_ref, ybar_ref, dinv_ref):
    deg = h0_ref[...] + h1_ref[...] + 1.0
    dinv = lax.rsqrt(deg)
    dinv_ref[...] = dinv
    ybar_ref[...] = dinv * x_ref[...]


_scale_call = pl.pallas_call(
    _scale_body,
    grid=(NP // 1024,),
    in_specs=[
        pl.BlockSpec((1024, 1), lambda i: (i, 0)),
        pl.BlockSpec((1024, 1), lambda i: (i, 0)),
        pl.BlockSpec((1024, D), lambda i: (i, 0)),
    ],
    out_specs=[
        pl.BlockSpec((1024, D), lambda i: (i, 0)),
        pl.BlockSpec((1024, 1), lambda i: (i, 0)),
    ],
    out_shape=[
        jax.ShapeDtypeStruct((NP, D), jnp.float32),
        jax.ShapeDtypeStruct((NP, 1), jnp.float32),
    ],
)


def _final_body(s0_ref, s1_ref, ybar_ref, dinv_ref, w_ref, out_ref):
    z = (s0_ref[...] + s1_ref[...] + ybar_ref[...]) * dinv_ref[...]
    out_ref[...] = lax.dot_general(
        z, w_ref[...], (((1,), (1,)), ((), ())),
        preferred_element_type=jnp.float32)


_final_call = pl.pallas_call(
    _final_body,
    grid=(NP // 1024,),
    in_specs=[
        pl.BlockSpec((1024, D), lambda i: (i, 0)),
        pl.BlockSpec((1024, D), lambda i: (i, 0)),
        pl.BlockSpec((1024, D), lambda i: (i, 0)),
        pl.BlockSpec((1024, 1), lambda i: (i, 0)),
        pl.BlockSpec((D, D), lambda i: (0, 0)),
    ],
    out_specs=pl.BlockSpec((1024, D), lambda i: (i, 0)),
    out_shape=jax.ShapeDtypeStruct((NP, D), jnp.float32),
)


def kernel(x, edge_index, W):
    ei = edge_index.astype(jnp.int32)
    row2d = ei[0].reshape(NW, NCH, CH)
    col2d = ei[1].reshape(NW, EPW)
    x_pad = jnp.pad(x, ((0, NP - N_NODES), (0, 0)))

    hist = _deg_kernel(row2d)
    ybar, dinv = _scale_call(hist[0].reshape(NP, 1), hist[1].reshape(NP, 1),
                             x_pad)
    spart = _agg_kernel(ybar, col2d, row2d)
    outp = _final_call(spart[0], spart[1], ybar, dinv, W)
    return outp[:N_NODES]
